# baseline (speedup 1.0000x reference)
import functools

import jax
import jax.numpy as jnp
from jax import lax
from jax.experimental import pallas as pl
from jax.experimental.pallas import tpu as pltpu
from jax.experimental.pallas import tpu_sc as plsc

VOCAB = 1000
HIDDEN = 128
BATCH = 1024
SEQ = 50

_VP = 1024
_NPAIR = _VP // 2
_NJT = 8

def _packed_body(pe_ref, po_ref, eb_ref, be_ref, bo_ref, out_ref):
    def half(p_ref, b_ref):
        t = lax.dot_general(
            p_ref[...], eb_ref[...],
            dimension_numbers=(((1,), (1,)), ((), ())),
            preferred_element_type=jnp.float32,
        ) + b_ref[...]
        u16 = lax.bitcast_convert_type(t.astype(jnp.bfloat16), jnp.uint16)
        return u16.astype(jnp.uint32)

    ue = half(pe_ref, be_ref)
    uo = half(po_ref, bo_ref)
    out_ref[...] = lax.bitcast_convert_type(ue | (uo << 16), jnp.int32)


def _build_packed_table(embed_weight, proj_weight, proj_bias):
    pe = jnp.pad(proj_weight[0::2], ((0, _NPAIR - 500), (0, 0)))
    po = jnp.pad(proj_weight[1::2], ((0, _NPAIR - 500), (0, 0)))
    eb = jnp.pad(embed_weight, ((0, _VP - VOCAB), (0, 0)))
    be = jnp.pad(proj_bias[0::2], (0, _NPAIR - 500)).reshape(_NPAIR, 1)
    bo = jnp.pad(proj_bias[1::2], (0, _NPAIR - 500)).reshape(_NPAIR, 1)
    x = pl.pallas_call(
        _packed_body,
        grid=(_NJT,),
        in_specs=[
            pl.BlockSpec((_NPAIR, HIDDEN), lambda jt: (0, 0)),
            pl.BlockSpec((_NPAIR, HIDDEN), lambda jt: (0, 0)),
            pl.BlockSpec((128, HIDDEN), lambda jt: (jt, 0)),
            pl.BlockSpec((_NPAIR, 1), lambda jt: (0, 0)),
            pl.BlockSpec((_NPAIR, 1), lambda jt: (0, 0)),
        ],
        out_specs=pl.BlockSpec((_NPAIR, 128), lambda jt: (jt, 0)),
        out_shape=jax.ShapeDtypeStruct((_NJT * _NPAIR, 128), jnp.int32),
    )(pe, po, eb, be, bo)
    return x.reshape(-1)


_INFO = plsc.get_sparse_core_info()
_NC, _NS = _INFO.num_cores, _INFO.num_subcores
_NW = _NC * _NS

_VT_PER_W = 4
_ROWS_W = _VT_PER_W * 8
_PAIRS_W = _ROWS_W // 2
_TBL_W = _NJT * _PAIRS_W * 128
_IDX_SPAN = (_NJT - 1) * _PAIRS_W * 128 + 128
_CHUNK_W = _ROWS_W * BATCH
_LAST_W = _NW - 1
_LAST_CHUNK = 8 * BATCH
_S_PLANE = VOCAB * BATCH
_OUT_WORDS = SEQ * _S_PLANE


@functools.partial(
    pl.kernel,
    mesh=plsc.VectorSubcoreMesh(core_axis_name="c", subcore_axis_name="s"),
    out_type=jax.ShapeDtypeStruct((_OUT_WORDS,), jnp.float32),
    scratch_types=[
        pltpu.VMEM((_TBL_W,), jnp.int32),
        pltpu.VMEM((BATCH,), jnp.int32),
        pltpu.VMEM((BATCH,), jnp.int32),
        pltpu.VMEM((_CHUNK_W,), jnp.float32),
        pltpu.VMEM((_CHUNK_W,), jnp.float32),
        pltpu.SemaphoreType.DMA,
        pltpu.SemaphoreType.DMA,
        pltpu.SemaphoreType.DMA,
        pltpu.SemaphoreType.DMA,
    ],
    compiler_params=pltpu.CompilerParams(
        use_tc_tiling_on_sc=False, needs_layout_passes=False),
)
def _gather_sc(table_hbm, ids_hbm, out_hbm,
               tbl_v, idx0, idx1, obuf0, obuf1,
               isem0, isem1, osem0, osem1):
    wid = lax.axis_index("s") * _NC + lax.axis_index("c")

    for jt in range(_NJT):
        pltpu.sync_copy(
            table_hbm.at[pl.ds(jt * _NPAIR * 128 + wid * _PAIRS_W * 128,
                               _PAIRS_W * 128)],
            tbl_v.at[pl.ds(jt * _PAIRS_W * 128, _PAIRS_W * 128)])

    def start_idx(s, buf, sem):
        return pltpu.async_copy(ids_hbm.at[s], buf, sem)

    def wait_idx(buf, sem):
        pltpu.make_async_copy(ids_hbm.at[0], buf, sem).wait()

    def start_out(s, buf, sem):
        off = pl.multiple_of(s * _S_PLANE + wid * _CHUNK_W, 1024)

        @pl.when(wid < _LAST_W)
        def _():
            pltpu.async_copy(buf, out_hbm.at[pl.ds(off, _CHUNK_W)], sem)

        @pl.when(wid == _LAST_W)
        def _():
            pltpu.async_copy(
                buf.at[pl.ds(0, _LAST_CHUNK)],
                out_hbm.at[pl.ds(off, _LAST_CHUNK)], sem)

    def wait_out(buf, sem):
        @pl.when(wid < _LAST_W)
        def _():
            pltpu.make_async_copy(
                out_hbm.at[pl.ds(0, _CHUNK_W)], buf, sem).wait()

        @pl.when(wid == _LAST_W)
        def _():
            pltpu.make_async_copy(
                out_hbm.at[pl.ds(0, _LAST_CHUNK)],
                buf.at[pl.ds(0, _LAST_CHUNK)], sem).wait()

    def compute(idx_ref, out_ref):
        @plsc.parallel_loop(0, 8, unroll=2)
        def bt_body(bt):
            ids = [
                idx_ref[pl.ds(pl.multiple_of(bt * 128 + bg * 16, 16), 16)]
                for bg in range(8)
            ]
            for bg in range(8):
                ids16 = ids[bg]
                for vp0 in range(0, _PAIRS_W, 8):
                    gs = [
                        plsc.load_gather(
                            tbl_v.at[pl.ds(vp * 128, _IDX_SPAN)], [ids16])
                        for vp in range(vp0, vp0 + 8)
                    ]
                    for i, vp in enumerate(range(vp0, vp0 + 8)):
                        lo, hi = plsc.unpack(
                            plsc.bitcast(gs[i], jnp.bfloat16),
                            format=plsc.PackFormat.INTERLEAVED,
                            preferred_element_type=jnp.float32)
                        for phase, vals in ((0, lo), (1, hi)):
                            vl = 2 * vp + phase
                            off = pl.multiple_of(
                                (vl // 8) * 8192 + bt * 1024
                                + (vl % 8) * 128 + bg * 16, 16)
                            out_ref[pl.ds(off, 16)] = vals

    start_idx(0, idx0, isem0)
    start_idx(1, idx1, isem1)

    def body(k, carry):
        s0 = 2 * k
        s1 = s0 + 1

        wait_idx(idx0, isem0)

        @pl.when(k > 0)
        def _():
            wait_out(obuf0, osem0)

        compute(idx0, obuf0)

        @pl.when(k < SEQ // 2 - 1)
        def _():
            start_idx(s0 + 2, idx0, isem0)

        start_out(s0, obuf0, osem0)

        wait_idx(idx1, isem1)

        @pl.when(k > 0)
        def _():
            wait_out(obuf1, osem1)

        compute(idx1, obuf1)

        @pl.when(k < SEQ // 2 - 1)
        def _():
            start_idx(s1 + 2, idx1, isem1)

        start_out(s1, obuf1, osem1)
        return carry

    lax.fori_loop(0, SEQ // 2, body, 0)
    wait_out(obuf0, osem0)
    wait_out(obuf1, osem1)


def kernel(input_ids, embed_weight, proj_weight, proj_bias):
    table = _build_packed_table(embed_weight, proj_weight, proj_bias)
    ids = input_ids.T.astype(jnp.int32)
    tids = ((ids >> 7) << 11) | (ids & 127)
    out1d = _gather_sc(table, tids)
    out5 = out1d.reshape(SEQ, VOCAB // 8, 8, 8, 128)
    return out5.transpose(2, 4, 0, 1, 3).reshape(BATCH, SEQ, VOCAB)


# speedup vs baseline: 1.0573x; 1.0573x over previous
import functools

import jax
import jax.numpy as jnp
from jax import lax
from jax.experimental import pallas as pl
from jax.experimental.pallas import tpu as pltpu
from jax.experimental.pallas import tpu_sc as plsc

VOCAB = 1000
HIDDEN = 128
BATCH = 1024
SEQ = 50

_VP = 1024
_NPAIR = _VP // 2
_NJT = 8

def _packed_body(proj_ref, eb_ref, bias_ref, out_ref):
    t = lax.dot_general(
        proj_ref[...], eb_ref[...],
        dimension_numbers=(((1,), (1,)), ((), ())),
        preferred_element_type=jnp.float32,
    ) + bias_ref[...]
    u = lax.bitcast_convert_type(
        t.astype(jnp.bfloat16), jnp.uint16).astype(jnp.uint32)
    u3 = u.reshape(_NPAIR, 2, 128)
    out_ref[...] = lax.bitcast_convert_type(
        u3[:, 0, :] | (u3[:, 1, :] << 16), jnp.int32)


def _build_packed_table(embed_weight, proj_weight, proj_bias):
    x = pl.pallas_call(
        _packed_body,
        grid=(_NJT,),
        in_specs=[
            pl.BlockSpec((_VP, HIDDEN), lambda jt: (0, 0)),
            pl.BlockSpec((128, HIDDEN), lambda jt: (jt, 0)),
            pl.BlockSpec((_VP, 1), lambda jt: (0, 0)),
        ],
        out_specs=pl.BlockSpec((_NPAIR, 128), lambda jt: (jt, 0)),
        out_shape=jax.ShapeDtypeStruct((_NJT * _NPAIR, 128), jnp.int32),
    )(proj_weight, embed_weight, proj_bias.reshape(VOCAB, 1))
    return x.reshape(-1)


_INFO = plsc.get_sparse_core_info()
_NC, _NS = _INFO.num_cores, _INFO.num_subcores
_NW = _NC * _NS

_VT_PER_W = 4
_ROWS_W = _VT_PER_W * 8
_PAIRS_W = _ROWS_W // 2
_TBL_W = _NJT * _PAIRS_W * 128
_IDX_SPAN = (_NJT - 1) * _PAIRS_W * 128 + 128
_CHUNK_W = _ROWS_W * BATCH
_LAST_W = _NW - 1
_LAST_CHUNK = 8 * BATCH
_S_PLANE = VOCAB * BATCH
_OUT_WORDS = SEQ * _S_PLANE


@functools.partial(
    pl.kernel,
    mesh=plsc.VectorSubcoreMesh(core_axis_name="c", subcore_axis_name="s"),
    out_type=jax.ShapeDtypeStruct((_OUT_WORDS,), jnp.float32),
    scratch_types=[
        pltpu.VMEM((_TBL_W,), jnp.int32),
        pltpu.VMEM((BATCH,), jnp.int32),
        pltpu.VMEM((BATCH,), jnp.int32),
        pltpu.VMEM((_CHUNK_W,), jnp.float32),
        pltpu.VMEM((_CHUNK_W,), jnp.float32),
        pltpu.SemaphoreType.DMA,
        pltpu.SemaphoreType.DMA,
        pltpu.SemaphoreType.DMA,
        pltpu.SemaphoreType.DMA,
    ],
    compiler_params=pltpu.CompilerParams(
        use_tc_tiling_on_sc=False, needs_layout_passes=False),
)
def _gather_sc(table_hbm, ids_hbm, out_hbm,
               tbl_v, idx0, idx1, obuf0, obuf1,
               isem0, isem1, osem0, osem1):
    wid = lax.axis_index("s") * _NC + lax.axis_index("c")

    for jt in range(_NJT):
        pltpu.sync_copy(
            table_hbm.at[pl.ds(jt * _NPAIR * 128 + wid * _PAIRS_W * 128,
                               _PAIRS_W * 128)],
            tbl_v.at[pl.ds(jt * _PAIRS_W * 128, _PAIRS_W * 128)])

    def start_idx(s, buf, sem):
        return pltpu.async_copy(ids_hbm.at[s], buf, sem)

    def wait_idx(buf, sem):
        pltpu.make_async_copy(ids_hbm.at[0], buf, sem).wait()

    def start_out(s, buf, sem):
        off = pl.multiple_of(s * _S_PLANE + wid * _CHUNK_W, 1024)

        @pl.when(wid < _LAST_W)
        def _():
            pltpu.async_copy(buf, out_hbm.at[pl.ds(off, _CHUNK_W)], sem)

        @pl.when(wid == _LAST_W)
        def _():
            pltpu.async_copy(
                buf.at[pl.ds(0, _LAST_CHUNK)],
                out_hbm.at[pl.ds(off, _LAST_CHUNK)], sem)

    def wait_out(buf, sem):
        @pl.when(wid < _LAST_W)
        def _():
            pltpu.make_async_copy(
                out_hbm.at[pl.ds(0, _CHUNK_W)], buf, sem).wait()

        @pl.when(wid == _LAST_W)
        def _():
            pltpu.make_async_copy(
                out_hbm.at[pl.ds(0, _LAST_CHUNK)],
                buf.at[pl.ds(0, _LAST_CHUNK)], sem).wait()

    def compute(idx_ref, out_ref):
        @plsc.parallel_loop(0, 8)
        def bt_body(bt):
            ids = [
                idx_ref[pl.ds(pl.multiple_of(bt * 128 + bg * 16, 16), 16)]
                for bg in range(8)
            ]
            for bg in range(8):
                ids16 = ids[bg]
                for vp0 in range(0, _PAIRS_W, 8):
                    gs = [
                        plsc.load_gather(
                            tbl_v.at[pl.ds(vp * 128, _IDX_SPAN)], [ids16])
                        for vp in range(vp0, vp0 + 8)
                    ]
                    for i, vp in enumerate(range(vp0, vp0 + 8)):
                        lo, hi = plsc.unpack(
                            plsc.bitcast(gs[i], jnp.bfloat16),
                            format=plsc.PackFormat.INTERLEAVED,
                            preferred_element_type=jnp.float32)
                        for phase, vals in ((0, lo), (1, hi)):
                            vl = 2 * vp + phase
                            off = pl.multiple_of(
                                (vl // 8) * 8192 + bt * 1024
                                + (vl % 8) * 128 + bg * 16, 16)
                            out_ref[pl.ds(off, 16)] = vals

    start_idx(0, idx0, isem0)
    start_idx(1, idx1, isem1)

    def body(k, carry):
        s0 = 2 * k
        s1 = s0 + 1

        wait_idx(idx0, isem0)

        @pl.when(k > 0)
        def _():
            wait_out(obuf0, osem0)

        compute(idx0, obuf0)

        @pl.when(k < SEQ // 2 - 1)
        def _():
            start_idx(s0 + 2, idx0, isem0)

        start_out(s0, obuf0, osem0)

        wait_idx(idx1, isem1)

        @pl.when(k > 0)
        def _():
            wait_out(obuf1, osem1)

        compute(idx1, obuf1)

        @pl.when(k < SEQ // 2 - 1)
        def _():
            start_idx(s1 + 2, idx1, isem1)

        start_out(s1, obuf1, osem1)
        return carry

    lax.fori_loop(0, SEQ // 2, body, 0)
    wait_out(obuf0, osem0)
    wait_out(obuf1, osem1)


def kernel(input_ids, embed_weight, proj_weight, proj_bias):
    table = _build_packed_table(embed_weight, proj_weight, proj_bias)
    ids = input_ids.T.astype(jnp.int32)
    tids = ((ids >> 7) << 11) | (ids & 127)
    out1d = _gather_sc(table, tids)
    out5 = out1d.reshape(SEQ, VOCAB // 8, 8, 8, 128)
    return out5.transpose(2, 4, 0, 1, 3).reshape(BATCH, SEQ, VOCAB)
